# baseline (speedup 1.0000x reference)
import functools
import math

import jax
import jax.numpy as jnp
from jax import lax
from jax.experimental import pallas as pl
from jax.experimental.pallas import tpu as pltpu
from jax.experimental.pallas import tpu_sc as plsc

D = 768
M = 16
B = 16
K = 2
EPS = 1e-08
COEFF = 0.01
N = 2048

BLK = 512
NBLK = N // BLK
NEG = -3.4028235e38

NW = 32
TPW = N // NW


CHUNKS = TPW // 16


def _sc_select(ht_hbm, ti_hbm, hs_hbm, tbuf, tibuf, hsbuf):
    wid = lax.axis_index("s") * 2 + lax.axis_index("c")

    pltpu.sync_copy(ht_hbm.at[wid], tbuf)

    for c in range(CHUNKS):
        sl = pl.ds(c * 16, 16)
        evs = []
        for m in range(M):
            sq = []
            for b in range(B):
                v = tbuf[m, pl.ds(b * TPW + c * 16, 16)]
                sq.append(v * v)
            half = B // 2
            while half >= 1:
                for j in range(half):
                    sq[j] = sq[j] + sq[j + half]
                half //= 2
            evs.append(sq[0])
        def tourney(vals):
            pairs = [(v, jnp.full((16,), m, jnp.int32))
                     for m, v in enumerate(vals)]
            while len(pairs) > 1:
                nxt = []
                for j in range(0, len(pairs), 2):
                    (va, ia), (vb, ib) = pairs[j], pairs[j + 1]
                    take = vb > va
                    nxt.append((jnp.where(take, vb, va),
                                jnp.where(take, ib, ia)))
                pairs = nxt
            return pairs[0]

        max1, idx1 = tourney(evs)
        mvecs = [jnp.full((16,), m, jnp.int32) for m in range(M)]
        m1f = [jnp.where(idx1 == mvecs[m], 1.0, 0.0) for m in range(M)]
        evs2 = [evs[m] - m1f[m] * 1e38 for m in range(M)]
        max2, idx2 = tourney(evs2)
        m2f = [jnp.where(idx2 == mvecs[m], 1.0, 0.0) for m in range(M)]
        tibuf[0, sl] = idx1
        tibuf[1, sl] = idx2

        for b in range(B):
            v0 = tbuf[0, pl.ds(b * TPW + c * 16, 16)]
            hs1 = v0 * m1f[0]
            hs2 = v0 * m2f[0]
            for m in range(1, M):
                v = tbuf[m, pl.ds(b * TPW + c * 16, 16)]
                hs1 = hs1 + v * m1f[m]
                hs2 = hs2 + v * m2f[m]
            hsbuf[b, sl] = hs1
            hsbuf[B + b, sl] = hs2

    pltpu.sync_copy(tibuf, ti_hbm.at[wid])
    pltpu.sync_copy(hsbuf, hs_hbm.at[wid])


_sc_select_call = functools.partial(
    pl.kernel,
    mesh=plsc.VectorSubcoreMesh(core_axis_name="c", subcore_axis_name="s"),
    out_type=[
        jax.ShapeDtypeStruct((NW, K, TPW), jnp.int32),
        jax.ShapeDtypeStruct((NW, K * B, TPW), jnp.float32),
    ],
    scratch_types=[
        pltpu.VMEM((M, B * TPW), jnp.float32),
        pltpu.VMEM((K, TPW), jnp.int32),
        pltpu.VMEM((K * B, TPW), jnp.float32),
    ],
)(_sc_select)


def _tc_finish(x_ref, h2_ref, v_ref, ti_ref, hs_ref, stats_ref, acc_ref,
               vacc_ref):
    i = pl.program_id(0)

    h2 = h2_ref[...]
    x = x_ref[...]
    ti = ti_ref[...]
    idx1 = ti[:, 0:1]
    idx2 = ti[:, 1:2]

    c_iota = jax.lax.broadcasted_iota(jnp.int32, (BLK, M * B), 1) // B
    fullmask = jnp.logical_or(c_iota == idx1, c_iota == idx2)
    h_masked = jnp.where(fullmask, h2, 0.0).astype(jnp.bfloat16)
    x_hat = jax.lax.dot_general(
        h_masked, v_ref[...].astype(jnp.bfloat16),
        dimension_numbers=(((1,), (1,)), ((), ())),
        preferred_element_type=jnp.float32)

    resid = x - x_hat
    hsb = hs_ref[...]

    hh_sum = jnp.sum(h2 * h2, axis=0, keepdims=True)
    seg = (jax.lax.broadcasted_iota(jnp.int32, (M * B, M), 0) // B
           == jax.lax.broadcasted_iota(jnp.int32, (M * B, M), 1)
           ).astype(jnp.float32)
    e_part = jax.lax.dot_general(
        hh_sum, seg, dimension_numbers=(((1,), (0,)), ((), ())),
        preferred_element_type=jnp.float32)

    m_iota = jax.lax.broadcasted_iota(jnp.int32, (BLK, M), 1)
    cnt = ((m_iota == idx1).astype(jnp.float32)
           + (m_iota == idx2).astype(jnp.float32))

    @pl.when(i == 0)
    def _init():
        acc_ref[0] = 0.0
        acc_ref[1] = 0.0
        acc_ref[2] = 0.0
        vacc_ref[...] = jnp.zeros_like(vacc_ref)

    acc_ref[0] += jnp.sum(resid * resid)
    acc_ref[1] += jnp.sum(x_hat * x_hat)
    acc_ref[2] += jnp.sum(hsb * hsb)
    vacc_ref[0:1, :] += e_part
    vacc_ref[1:2, :] += jnp.sum(cnt, axis=0, keepdims=True)

    @pl.when(i == NBLK - 1)
    def _finalize():
        inv_n = 1.0 / N
        uncaptured = acc_ref[0] * inv_n
        recon = acc_ref[1] * inv_n
        captured = acc_ref[2] * inv_n
        counts = vacc_ref[1:2, :]
        avg_e = vacc_ref[0:1, :] * inv_n
        denom = jnp.maximum(jnp.sum(avg_e), EPS)
        probs = jnp.maximum(avg_e / denom, EPS)
        be = -jnp.sum(probs * jnp.log(probs)) / math.log(float(M))
        expected = K / float(M) * float(N)
        nlow = jnp.sum((counts <= 0.1 * expected).astype(jnp.float32))
        ndead = jnp.sum((counts <= 0.01 * expected).astype(jnp.float32))
        stats_ref[0] = captured
        stats_ref[1] = recon
        stats_ref[2] = uncaptured
        stats_ref[3] = be
        stats_ref[4] = uncaptured + COEFF * (1.0 - be)
        stats_ref[5] = nlow
        stats_ref[6] = ndead


def kernel(x_flat, h_all, V):
    h2 = h_all.reshape(N, M * B)
    hw = (h_all.reshape(NW, TPW, M, B)
          .transpose(0, 2, 3, 1)
          .reshape(NW, M, B * TPW))
    v2 = V.reshape(D, M * B)

    ti_w, hs_w = _sc_select_call(hw)

    ti = ti_w.transpose(0, 2, 1).reshape(N, K)
    hs2 = hs_w.transpose(0, 2, 1).reshape(N, K * B)

    stats, = pl.pallas_call(
        _tc_finish,
        grid=(NBLK,),
        in_specs=[
            pl.BlockSpec((BLK, D), lambda i: (i, 0)),
            pl.BlockSpec((BLK, M * B), lambda i: (i, 0)),
            pl.BlockSpec((D, M * B), lambda i: (0, 0)),
            pl.BlockSpec((BLK, K), lambda i: (i, 0)),
            pl.BlockSpec((BLK, K * B), lambda i: (i, 0)),
        ],
        out_specs=[
            pl.BlockSpec(memory_space=pltpu.SMEM),
        ],
        out_shape=[
            jax.ShapeDtypeStruct((8,), jnp.float32),
        ],
        scratch_shapes=[
            pltpu.SMEM((4,), jnp.float32),
            pltpu.VMEM((2, M), jnp.float32),
        ],
    )(x_flat, h2, v2, ti, hs2)

    return (hs2.reshape(N, K, B), ti, stats[0], stats[1], stats[2],
            stats[3], stats[4], stats[5], stats[6])


# speedup vs baseline: 1.0745x; 1.0745x over previous
import functools
import math

import jax
import jax.numpy as jnp
from jax import lax
from jax.experimental import pallas as pl
from jax.experimental.pallas import tpu as pltpu
from jax.experimental.pallas import tpu_sc as plsc

D = 768
M = 16
B = 16
K = 2
EPS = 1e-08
COEFF = 0.01
N = 2048

BLK = 512
NBLK = N // BLK
NEG = -3.4028235e38

NW = 32
TPW = N // NW


CHUNKS = TPW // 16


def _sc_select(ht_hbm, ti_hbm, hs_hbm, tbuf, tibuf, hsbuf):
    s = lax.axis_index("s")
    core = lax.axis_index("c")
    wid = s * 2 + core
    off = core * TPW

    pltpu.sync_copy(ht_hbm.at[:, :, pl.ds(s * 2 * TPW, 2 * TPW)], tbuf)

    for c in range(CHUNKS):
        sl = pl.ds(c * 16, 16)
        evs = []
        for m in range(M):
            sq = []
            for b in range(B):
                v = tbuf[m, b, pl.ds(off + c * 16, 16)]
                sq.append(v * v)
            half = B // 2
            while half >= 1:
                for j in range(half):
                    sq[j] = sq[j] + sq[j + half]
                half //= 2
            evs.append(sq[0])
        def tourney(vals):
            pairs = [(v, jnp.full((16,), m, jnp.int32))
                     for m, v in enumerate(vals)]
            while len(pairs) > 1:
                nxt = []
                for j in range(0, len(pairs), 2):
                    (va, ia), (vb, ib) = pairs[j], pairs[j + 1]
                    take = vb > va
                    nxt.append((jnp.where(take, vb, va),
                                jnp.where(take, ib, ia)))
                pairs = nxt
            return pairs[0]

        max1, idx1 = tourney(evs)
        mvecs = [jnp.full((16,), m, jnp.int32) for m in range(M)]
        m1f = [jnp.where(idx1 == mvecs[m], 1.0, 0.0) for m in range(M)]
        evs2 = [evs[m] - m1f[m] * 1e38 for m in range(M)]
        max2, idx2 = tourney(evs2)
        m2f = [jnp.where(idx2 == mvecs[m], 1.0, 0.0) for m in range(M)]
        tibuf[0, sl] = idx1
        tibuf[1, sl] = idx2

        for b in range(B):
            v0 = tbuf[0, b, pl.ds(off + c * 16, 16)]
            hs1 = v0 * m1f[0]
            hs2 = v0 * m2f[0]
            for m in range(1, M):
                v = tbuf[m, b, pl.ds(off + c * 16, 16)]
                hs1 = hs1 + v * m1f[m]
                hs2 = hs2 + v * m2f[m]
            hsbuf[b, sl] = hs1
            hsbuf[B + b, sl] = hs2

    pltpu.sync_copy(tibuf, ti_hbm.at[wid])
    pltpu.sync_copy(hsbuf, hs_hbm.at[wid])


_sc_select_call = functools.partial(
    pl.kernel,
    mesh=plsc.VectorSubcoreMesh(core_axis_name="c", subcore_axis_name="s"),
    out_type=[
        jax.ShapeDtypeStruct((NW, K, TPW), jnp.int32),
        jax.ShapeDtypeStruct((NW, K * B, TPW), jnp.float32),
    ],
    scratch_types=[
        pltpu.VMEM((M, B, 2 * TPW), jnp.float32),
        pltpu.VMEM((K, TPW), jnp.int32),
        pltpu.VMEM((K * B, TPW), jnp.float32),
    ],
)(_sc_select)


def _tc_finish(x_ref, h2_ref, v_ref, ti_ref, hs_ref, stats_ref, acc_ref,
               vacc_ref):
    i = pl.program_id(0)

    h2 = h2_ref[...]
    x = x_ref[...]
    ti = ti_ref[...]
    idx1 = ti[:, 0:1]
    idx2 = ti[:, 1:2]

    c_iota = jax.lax.broadcasted_iota(jnp.int32, (BLK, M * B), 1) // B
    fullmask = jnp.logical_or(c_iota == idx1, c_iota == idx2)
    h_masked = jnp.where(fullmask, h2, 0.0).astype(jnp.bfloat16)
    x_hat = jax.lax.dot_general(
        h_masked, v_ref[...].astype(jnp.bfloat16),
        dimension_numbers=(((1,), (1,)), ((), ())),
        preferred_element_type=jnp.float32)

    resid = x - x_hat
    hsb = hs_ref[...]

    hh_sum = jnp.sum(h2 * h2, axis=0, keepdims=True)
    seg = (jax.lax.broadcasted_iota(jnp.int32, (M * B, M), 0) // B
           == jax.lax.broadcasted_iota(jnp.int32, (M * B, M), 1)
           ).astype(jnp.float32)
    e_part = jax.lax.dot_general(
        hh_sum, seg, dimension_numbers=(((1,), (0,)), ((), ())),
        preferred_element_type=jnp.float32)

    m_iota = jax.lax.broadcasted_iota(jnp.int32, (BLK, M), 1)
    cnt = ((m_iota == idx1).astype(jnp.float32)
           + (m_iota == idx2).astype(jnp.float32))

    @pl.when(i == 0)
    def _init():
        acc_ref[0] = 0.0
        acc_ref[1] = 0.0
        acc_ref[2] = 0.0
        vacc_ref[...] = jnp.zeros_like(vacc_ref)

    acc_ref[0] += jnp.sum(resid * resid)
    acc_ref[1] += jnp.sum(x_hat * x_hat)
    acc_ref[2] += jnp.sum(hsb * hsb)
    vacc_ref[0:1, :] += e_part
    vacc_ref[1:2, :] += jnp.sum(cnt, axis=0, keepdims=True)

    @pl.when(i == NBLK - 1)
    def _finalize():
        inv_n = 1.0 / N
        uncaptured = acc_ref[0] * inv_n
        recon = acc_ref[1] * inv_n
        captured = acc_ref[2] * inv_n
        counts = vacc_ref[1:2, :]
        avg_e = vacc_ref[0:1, :] * inv_n
        denom = jnp.maximum(jnp.sum(avg_e), EPS)
        probs = jnp.maximum(avg_e / denom, EPS)
        be = -jnp.sum(probs * jnp.log(probs)) / math.log(float(M))
        expected = K / float(M) * float(N)
        nlow = jnp.sum((counts <= 0.1 * expected).astype(jnp.float32))
        ndead = jnp.sum((counts <= 0.01 * expected).astype(jnp.float32))
        stats_ref[0] = captured
        stats_ref[1] = recon
        stats_ref[2] = uncaptured
        stats_ref[3] = be
        stats_ref[4] = uncaptured + COEFF * (1.0 - be)
        stats_ref[5] = nlow
        stats_ref[6] = ndead


def kernel(x_flat, h_all, V):
    h2 = h_all.reshape(N, M * B)
    hw = jnp.transpose(h_all, (1, 2, 0))
    v2 = V.reshape(D, M * B)

    ti_w, hs_w = _sc_select_call(hw)

    ti = ti_w.transpose(0, 2, 1).reshape(N, K)
    hs2 = hs_w.transpose(0, 2, 1).reshape(N, K * B)

    stats, = pl.pallas_call(
        _tc_finish,
        grid=(NBLK,),
        in_specs=[
            pl.BlockSpec((BLK, D), lambda i: (i, 0)),
            pl.BlockSpec((BLK, M * B), lambda i: (i, 0)),
            pl.BlockSpec((D, M * B), lambda i: (0, 0)),
            pl.BlockSpec((BLK, K), lambda i: (i, 0)),
            pl.BlockSpec((BLK, K * B), lambda i: (i, 0)),
        ],
        out_specs=[
            pl.BlockSpec(memory_space=pltpu.SMEM),
        ],
        out_shape=[
            jax.ShapeDtypeStruct((8,), jnp.float32),
        ],
        scratch_shapes=[
            pltpu.SMEM((4,), jnp.float32),
            pltpu.VMEM((2, M), jnp.float32),
        ],
    )(x_flat, h2, v2, ti, hs2)

    return (hs2.reshape(N, K, B), ti, stats[0], stats[1], stats[2],
            stats[3], stats[4], stats[5], stats[6])


# speedup vs baseline: 1.0777x; 1.0030x over previous
import functools
import math

import jax
import jax.numpy as jnp
from jax import lax
from jax.experimental import pallas as pl
from jax.experimental.pallas import tpu as pltpu
from jax.experimental.pallas import tpu_sc as plsc

D = 768
M = 16
B = 16
K = 2
EPS = 1e-08
COEFF = 0.01
N = 2048

BLK = 512
NBLK = N // BLK
NEG = -3.4028235e38

NW = 32
TPW = N // NW


CHUNKS = TPW // 16


def _sc_select(ht_hbm, ti_hbm, hs_hbm, tbuf, tibuf, hsbuf):
    s = lax.axis_index("s")
    core = lax.axis_index("c")
    wid = s * 2 + core
    off = core * TPW

    pltpu.sync_copy(ht_hbm.at[:, :, pl.ds(s * 2 * TPW, 2 * TPW)], tbuf)
    tview = tbuf.at[:, :, pl.ds(off, TPW)]

    for c in range(CHUNKS):
        sl = pl.ds(c * 16, 16)
        evs = []
        for m in range(M):
            sq = []
            for b in range(B):
                v = tview[m, b, pl.ds(c * 16, 16)]
                sq.append(v * v)
            half = B // 2
            while half >= 1:
                for j in range(half):
                    sq[j] = sq[j] + sq[j + half]
                half //= 2
            evs.append(sq[0])
        def tourney(vals):
            pairs = [(v, jnp.full((16,), m, jnp.int32))
                     for m, v in enumerate(vals)]
            while len(pairs) > 1:
                nxt = []
                for j in range(0, len(pairs), 2):
                    (va, ia), (vb, ib) = pairs[j], pairs[j + 1]
                    take = vb > va
                    nxt.append((jnp.where(take, vb, va),
                                jnp.where(take, ib, ia)))
                pairs = nxt
            return pairs[0]

        max1, idx1 = tourney(evs)
        mvecs = [jnp.full((16,), m, jnp.int32) for m in range(M)]
        m1f = [jnp.where(idx1 == mvecs[m], 1.0, 0.0) for m in range(M)]
        evs2 = [evs[m] - m1f[m] * 1e38 for m in range(M)]
        max2, idx2 = tourney(evs2)
        m2f = [jnp.where(idx2 == mvecs[m], 1.0, 0.0) for m in range(M)]
        tibuf[0, sl] = idx1
        tibuf[1, sl] = idx2

        for b in range(B):
            v0 = tview[0, b, pl.ds(c * 16, 16)]
            hs1 = v0 * m1f[0]
            hs2 = v0 * m2f[0]
            for m in range(1, M):
                v = tview[m, b, pl.ds(c * 16, 16)]
                hs1 = hs1 + v * m1f[m]
                hs2 = hs2 + v * m2f[m]
            hsbuf[b, sl] = hs1
            hsbuf[B + b, sl] = hs2

    pltpu.sync_copy(tibuf, ti_hbm.at[wid])
    pltpu.sync_copy(hsbuf, hs_hbm.at[wid])


_sc_select_call = functools.partial(
    pl.kernel,
    mesh=plsc.VectorSubcoreMesh(core_axis_name="c", subcore_axis_name="s"),
    out_type=[
        jax.ShapeDtypeStruct((NW, K, TPW), jnp.int32),
        jax.ShapeDtypeStruct((NW, K * B, TPW), jnp.float32),
    ],
    scratch_types=[
        pltpu.VMEM((M, B, 2 * TPW), jnp.float32),
        pltpu.VMEM((K, TPW), jnp.int32),
        pltpu.VMEM((K * B, TPW), jnp.float32),
    ],
)(_sc_select)


def _tc_finish(x_ref, h2_ref, v_ref, ti_ref, hs_ref, stats_ref, acc_ref,
               vacc_ref):
    i = pl.program_id(0)

    h2 = h2_ref[...]
    x = x_ref[...]
    ti = ti_ref[...]
    idx1 = ti[:, 0:1]
    idx2 = ti[:, 1:2]

    c_iota = jax.lax.broadcasted_iota(jnp.int32, (BLK, M * B), 1) // B
    fullmask = jnp.logical_or(c_iota == idx1, c_iota == idx2)
    h_masked = jnp.where(fullmask, h2, 0.0).astype(jnp.bfloat16)
    x_hat = jax.lax.dot_general(
        h_masked, v_ref[...].astype(jnp.bfloat16),
        dimension_numbers=(((1,), (1,)), ((), ())),
        preferred_element_type=jnp.float32)

    resid = x - x_hat
    hsb = hs_ref[...]

    hh_sum = jnp.sum(h2 * h2, axis=0, keepdims=True)
    seg = (jax.lax.broadcasted_iota(jnp.int32, (M * B, M), 0) // B
           == jax.lax.broadcasted_iota(jnp.int32, (M * B, M), 1)
           ).astype(jnp.float32)
    e_part = jax.lax.dot_general(
        hh_sum, seg, dimension_numbers=(((1,), (0,)), ((), ())),
        preferred_element_type=jnp.float32)

    m_iota = jax.lax.broadcasted_iota(jnp.int32, (BLK, M), 1)
    cnt = ((m_iota == idx1).astype(jnp.float32)
           + (m_iota == idx2).astype(jnp.float32))

    @pl.when(i == 0)
    def _init():
        acc_ref[0] = 0.0
        acc_ref[1] = 0.0
        acc_ref[2] = 0.0
        vacc_ref[...] = jnp.zeros_like(vacc_ref)

    acc_ref[0] += jnp.sum(resid * resid)
    acc_ref[1] += jnp.sum(x_hat * x_hat)
    acc_ref[2] += jnp.sum(hsb * hsb)
    vacc_ref[0:1, :] += e_part
    vacc_ref[1:2, :] += jnp.sum(cnt, axis=0, keepdims=True)

    @pl.when(i == NBLK - 1)
    def _finalize():
        inv_n = 1.0 / N
        uncaptured = acc_ref[0] * inv_n
        recon = acc_ref[1] * inv_n
        captured = acc_ref[2] * inv_n
        counts = vacc_ref[1:2, :]
        avg_e = vacc_ref[0:1, :] * inv_n
        denom = jnp.maximum(jnp.sum(avg_e), EPS)
        probs = jnp.maximum(avg_e / denom, EPS)
        be = -jnp.sum(probs * jnp.log(probs)) / math.log(float(M))
        expected = K / float(M) * float(N)
        nlow = jnp.sum((counts <= 0.1 * expected).astype(jnp.float32))
        ndead = jnp.sum((counts <= 0.01 * expected).astype(jnp.float32))
        stats_ref[0] = captured
        stats_ref[1] = recon
        stats_ref[2] = uncaptured
        stats_ref[3] = be
        stats_ref[4] = uncaptured + COEFF * (1.0 - be)
        stats_ref[5] = nlow
        stats_ref[6] = ndead


def kernel(x_flat, h_all, V):
    h2 = h_all.reshape(N, M * B)
    hw = jnp.transpose(h_all, (1, 2, 0))
    v2 = V.reshape(D, M * B)

    ti_w, hs_w = _sc_select_call(hw)

    ti = ti_w.transpose(0, 2, 1).reshape(N, K)
    hs2 = hs_w.transpose(0, 2, 1).reshape(N, K * B)

    stats, = pl.pallas_call(
        _tc_finish,
        grid=(NBLK,),
        in_specs=[
            pl.BlockSpec((BLK, D), lambda i: (i, 0)),
            pl.BlockSpec((BLK, M * B), lambda i: (i, 0)),
            pl.BlockSpec((D, M * B), lambda i: (0, 0)),
            pl.BlockSpec((BLK, K), lambda i: (i, 0)),
            pl.BlockSpec((BLK, K * B), lambda i: (i, 0)),
        ],
        out_specs=[
            pl.BlockSpec(memory_space=pltpu.SMEM),
        ],
        out_shape=[
            jax.ShapeDtypeStruct((8,), jnp.float32),
        ],
        scratch_shapes=[
            pltpu.SMEM((4,), jnp.float32),
            pltpu.VMEM((2, M), jnp.float32),
        ],
    )(x_flat, h2, v2, ti, hs2)

    return (hs2.reshape(N, K, B), ti, stats[0], stats[1], stats[2],
            stats[3], stats[4], stats[5], stats[6])
